# baseline (speedup 1.0000x reference)
import functools

import numpy as np
import jax
import jax.numpy as jnp
from jax import lax
from jax.experimental import pallas as pl
from jax.experimental.pallas import tpu as pltpu
from jax.experimental.pallas import tpu_sc as plsc

R, V = 128, 100000
L = 16
NG = R // L
SLICE = 1792
CW = 208
SCV = 32 * SLICE
TCB = 4096
TCN = -(-(V - SCV) // TCB)
NACC = 32

_CONST_CACHE = None


def _np_threefry2x32(k0, k1, x0, x1):
    rot0 = (13, 15, 26, 6)
    rot1 = (17, 29, 16, 24)
    ks0 = np.uint32(k0)
    ks1 = np.uint32(k1)
    ks2 = np.uint32(ks0 ^ ks1 ^ np.uint32(0x1BD11BDA))

    def rotl(x, d):
        return (x << np.uint32(d)) | (x >> np.uint32(32 - d))

    x0 = x0 + ks0
    x1 = x1 + ks1
    keys = [(ks1, ks2), (ks2, ks0), (ks0, ks1), (ks1, ks2), (ks2, ks0)]
    rots = [rot0, rot1, rot0, rot1, rot0]
    for i in range(5):
        for r in rots[i]:
            x0 = x0 + x1
            x1 = rotl(x1, r)
            x1 = x1 ^ x0
        a, b = keys[i]
        x0 = x0 + a
        x1 = x1 + b + np.uint32(i + 1)
    return x0, x1


def _const_table():
    global _CONST_CACHE
    if _CONST_CACHE is None:
        n = R * V
        hi = np.zeros(n, dtype=np.uint32)
        lo = np.arange(n, dtype=np.uint32)
        with np.errstate(over="ignore"):
            b0, b1 = _np_threefry2x32(np.uint32(0), np.uint32(42), hi, lo)
        bits = b0 ^ b1
        u = ((bits >> np.uint32(9)) | np.uint32(0x3F800000)).view(np.float32)
        u = u - np.float32(1.0)
        noise = (-np.log1p(-u)).astype(np.float32)
        noise = np.maximum(noise, np.float32(1e-10))
        c = (-np.log(noise)).astype(np.float32).reshape(R, V)
        ct = np.ascontiguousarray(c.T)
        _CONST_CACHE = (np.ascontiguousarray(ct[:SCV]),
                        np.ascontiguousarray(ct[SCV:]))
    return tuple(jnp.asarray(x) for x in _CONST_CACHE)


def _sampler_body(logits_hbm, c_hbm, invt_hbm, g_hbm, outv_hbm, outi_hbm,
                  xb0, xb1, cb0, cb1, pt, pg, sbf, sbi, mfb, mib,
                  vbuf, ibuf, shf, shi, sem0, sem1):
    cid = lax.axis_index("c")
    sid = lax.axis_index("s")
    wid = cid * 16 + sid
    lane = lax.iota(jnp.int32, L)

    pltpu.sync_copy(invt_hbm, pt)
    pltpu.sync_copy(g_hbm, pg)
    invTs = [pt[pl.ds(b * L, L)] for b in range(NG)]
    gs = [pg[pl.ds(b * L, L)] for b in range(NG)]

    base = wid * SLICE
    xbufs = (xb0, xb1)
    cbufs = (cb0, cb1)
    sems = (sem0, sem1)
    nfull = SLICE // CW
    chunks = [(base, k * CW, CW) for k in range(nfull)]
    if SLICE % CW:
        chunks.append((base, nfull * CW, SLICE % CW))

    def start(k, buf):
        b0, off, w = chunks[k]
        voff = pl.multiple_of(b0 + off, 8)
        hx = pltpu.async_copy(logits_hbm.at[pl.ds(voff, w), :],
                              xbufs[buf].at[pl.ds(0, w), :], sems[buf])
        hc = pltpu.async_copy(c_hbm.at[pl.ds(voff, w), :],
                              cbufs[buf].at[pl.ds(0, w), :], sems[buf])
        return hx, hc

    handles = [None, None]
    handles[0] = start(0, 0)

    ninf = jnp.full((L,), -jnp.inf, jnp.float32)
    zero = jnp.zeros((L,), jnp.int32)
    bvs = [ninf] * NG
    bis = [zero] * NG

    for k in range(len(chunks)):
        cur = k % 2
        hx, hc = handles[cur]
        hx.wait()
        hc.wait()
        if k + 1 < len(chunks):
            handles[1 - cur] = start(k + 1, 1 - cur)
        xref, cref = xbufs[cur], cbufs[cur]
        b0, off, w = chunks[k]
        v0 = b0 + off

        def body(i, carry, xref=xref, cref=cref):
            st = list(carry[:2 * NG])
            iv = carry[2 * NG]
            for b in range(NG):
                x = xref[i, pl.ds(b * L, L)]
                c = cref[i, pl.ds(b * L, L)]
                s = x * invTs[b] + gs[b] * c
                m = s > st[b]
                st[b] = jnp.where(m, s, st[b])
                st[NG + b] = jnp.where(m, iv, st[NG + b])
            return tuple(st) + (iv + 1,)

        iv0 = jnp.broadcast_to(jnp.int32(0), (L,)) + v0
        out = lax.fori_loop(0, w, body, tuple(bvs) + tuple(bis) + (iv0,))
        bvs = list(out[:NG])
        bis = list(out[NG:2 * NG])

    for b in range(NG):
        sbf[pl.ds(b * L, L)] = bvs[b]
        sbi[pl.ds(b * L, L)] = bis[b]
    pltpu.sync_copy(sbf, shf.at[pl.ds(sid * R, R)])
    pltpu.sync_copy(sbi, shi.at[pl.ds(sid * R, R)])
    plsc.subcore_barrier()

    @pl.when(sid == 0)
    def _():
        pltpu.sync_copy(shf, mfb)
        pltpu.sync_copy(shi, mib)
        for b in range(NG):
            av = mfb[pl.ds(b * L, L)]
            ai = mib[pl.ds(b * L, L)]
            for s2 in range(1, 16):
                ov = mfb[pl.ds(s2 * R + b * L, L)]
                oi = mib[pl.ds(s2 * R + b * L, L)]
                m = (ov > av) | ((ov == av) & (oi < ai))
                av = jnp.where(m, ov, av)
                ai = jnp.where(m, oi, ai)
            vbuf[b, pl.ds(0, L)] = av
            ibuf[b, pl.ds(0, L)] = ai
        ro = pl.multiple_of(cid * NG, 8)
        pltpu.sync_copy(vbuf, outv_hbm.at[pl.ds(ro, NG), :])
        pltpu.sync_copy(ibuf, outi_hbm.at[pl.ds(ro, NG), :])


_sampler = functools.partial(
    pl.kernel,
    out_type=(jax.ShapeDtypeStruct((2 * NG, 128), jnp.float32),
              jax.ShapeDtypeStruct((2 * NG, 128), jnp.int32)),
    mesh=plsc.VectorSubcoreMesh(core_axis_name="c", subcore_axis_name="s"),
    compiler_params=pltpu.CompilerParams(use_tc_tiling_on_sc=True),
    scratch_types=[
        pltpu.VMEM((CW, 128), jnp.float32),
        pltpu.VMEM((CW, 128), jnp.float32),
        pltpu.VMEM((CW, 128), jnp.float32),
        pltpu.VMEM((CW, 128), jnp.float32),
        pltpu.VMEM((R,), jnp.float32),
        pltpu.VMEM((R,), jnp.float32),
        pltpu.VMEM((R,), jnp.float32),
        pltpu.VMEM((R,), jnp.int32),
        pltpu.VMEM((16 * R,), jnp.float32),
        pltpu.VMEM((16 * R,), jnp.int32),
        pltpu.VMEM((NG, 128), jnp.float32),
        pltpu.VMEM((NG, 128), jnp.int32),
        pltpu.VMEM_SHARED((16 * R,), jnp.float32),
        pltpu.VMEM_SHARED((16 * R,), jnp.int32),
        pltpu.SemaphoreType.DMA,
        pltpu.SemaphoreType.DMA,
    ],
)(_sampler_body)


def _tc_body(invt_ref, g_ref, x_ref, c_ref, outv_ref, outi_ref, rv, ri):
    k = pl.program_id(0)

    @pl.when(k == 0)
    def _():
        rv[...] = jnp.full((NACC * 8, 128), -jnp.inf, jnp.float32)
        ri[...] = jnp.zeros((NACC * 8, 128), jnp.int32)

    s = x_ref[...] * invt_ref[...] + g_ref[...] * c_ref[...]
    base = SCV + k * TCB
    rows = jax.lax.broadcasted_iota(jnp.int32, (TCB, 128), 0) + base
    rvv = [rv[pl.ds(a * 8, 8), :] for a in range(NACC)]
    riv = [ri[pl.ds(a * 8, 8), :] for a in range(NACC)]
    for j in range(TCB // 8):
        a = j % NACC
        sub = s[j * 8:(j + 1) * 8]
        idx = rows[j * 8:(j + 1) * 8]
        m = (sub > rvv[a]) & (idx < V)
        rvv[a] = jnp.where(m, sub, rvv[a])
        riv[a] = jnp.where(m, idx, riv[a])
    for a in range(NACC):
        rv[pl.ds(a * 8, 8), :] = rvv[a]
        ri[pl.ds(a * 8, 8), :] = riv[a]

    @pl.when(k == TCN - 1)
    def _():
        fv, fi = rvv[0], riv[0]
        for a in range(1, NACC):
            m = (rvv[a] > fv) | ((rvv[a] == fv) & (riv[a] < fi))
            fv = jnp.where(m, rvv[a], fv)
            fi = jnp.where(m, riv[a], fi)
        vmax = jnp.max(fv, axis=0, keepdims=True)
        cand = jnp.where(fv == vmax, fi, V)
        outv_ref[...] = vmax
        outi_ref[...] = jnp.min(cand, axis=0, keepdims=True)


_tc_argmax = pl.pallas_call(
    _tc_body,
    grid=(TCN,),
    in_specs=[
        pl.BlockSpec((1, 128), lambda k: (0, 0)),
        pl.BlockSpec((1, 128), lambda k: (0, 0)),
        pl.BlockSpec((TCB, 128), lambda k: (SCV // TCB + k, 0)),
        pl.BlockSpec((TCB, 128), lambda k: (k, 0)),
    ],
    out_specs=[
        pl.BlockSpec((1, 128), lambda k: (0, 0)),
        pl.BlockSpec((1, 128), lambda k: (0, 0)),
    ],
    out_shape=[
        jax.ShapeDtypeStruct((1, 128), jnp.float32),
        jax.ShapeDtypeStruct((1, 128), jnp.int32),
    ],
    scratch_shapes=[
        pltpu.VMEM((NACC * 8, 128), jnp.float32),
        pltpu.VMEM((NACC * 8, 128), jnp.int32),
    ],
)


def _lex_merge(v0, i0, v1, i1):
    take = (v1 > v0) | ((v1 == v0) & (i1 < i0))
    return jnp.where(take, v1, v0), jnp.where(take, i1, i0)


def kernel(logits, temperatures):
    c_sc, c_tc = _const_table()
    sampled = temperatures > 1e-10
    inv_t = jnp.where(sampled, 1.0 / jnp.where(sampled, temperatures, 1.0), 1.0)
    g = sampled.astype(jnp.float32)
    lt = logits.astype(jnp.float32).T
    outv, outi = _sampler(lt, c_sc, inv_t, g)
    tcv, tci = _tc_argmax(inv_t[None, :], g[None, :], lt, c_tc)
    mv, mi = _lex_merge(outv[0:NG, 0:L].reshape(R), outi[0:NG, 0:L].reshape(R),
                        outv[NG:, 0:L].reshape(R), outi[NG:, 0:L].reshape(R))
    _, ti = _lex_merge(mv, mi, tcv.reshape(R), tci.reshape(R))
    return ti


# speedup vs baseline: 1.0140x; 1.0140x over previous
import functools

import numpy as np
import jax
import jax.numpy as jnp
from jax import lax
from jax.experimental import pallas as pl
from jax.experimental.pallas import tpu as pltpu
from jax.experimental.pallas import tpu_sc as plsc

R, V = 128, 100000
L = 16
NG = R // L
SLICE = 1792
CW = 208
SCV = 32 * SLICE
TCB = 2048
TCN = -(-(V - SCV) // TCB)
NACC = 16

_CONST_CACHE = None


def _np_threefry2x32(k0, k1, x0, x1):
    rot0 = (13, 15, 26, 6)
    rot1 = (17, 29, 16, 24)
    ks0 = np.uint32(k0)
    ks1 = np.uint32(k1)
    ks2 = np.uint32(ks0 ^ ks1 ^ np.uint32(0x1BD11BDA))

    def rotl(x, d):
        return (x << np.uint32(d)) | (x >> np.uint32(32 - d))

    x0 = x0 + ks0
    x1 = x1 + ks1
    keys = [(ks1, ks2), (ks2, ks0), (ks0, ks1), (ks1, ks2), (ks2, ks0)]
    rots = [rot0, rot1, rot0, rot1, rot0]
    for i in range(5):
        for r in rots[i]:
            x0 = x0 + x1
            x1 = rotl(x1, r)
            x1 = x1 ^ x0
        a, b = keys[i]
        x0 = x0 + a
        x1 = x1 + b + np.uint32(i + 1)
    return x0, x1


def _const_table():
    global _CONST_CACHE
    if _CONST_CACHE is None:
        n = R * V
        hi = np.zeros(n, dtype=np.uint32)
        lo = np.arange(n, dtype=np.uint32)
        with np.errstate(over="ignore"):
            b0, b1 = _np_threefry2x32(np.uint32(0), np.uint32(42), hi, lo)
        bits = b0 ^ b1
        u = ((bits >> np.uint32(9)) | np.uint32(0x3F800000)).view(np.float32)
        u = u - np.float32(1.0)
        noise = (-np.log1p(-u)).astype(np.float32)
        noise = np.maximum(noise, np.float32(1e-10))
        c = (-np.log(noise)).astype(np.float32).reshape(R, V)
        ct = np.ascontiguousarray(c.T)
        _CONST_CACHE = (np.ascontiguousarray(ct[:SCV]),
                        np.ascontiguousarray(ct[SCV:]))
    return tuple(jnp.asarray(x) for x in _CONST_CACHE)


def _sampler_body(logits_hbm, c_hbm, invt_hbm, g_hbm, outv_hbm, outi_hbm,
                  xb0, xb1, cb0, cb1, pt, pg, sbf, sbi, mfb, mib,
                  vbuf, ibuf, shf, shi, sem0, sem1):
    cid = lax.axis_index("c")
    sid = lax.axis_index("s")
    wid = cid * 16 + sid
    lane = lax.iota(jnp.int32, L)

    pltpu.sync_copy(invt_hbm, pt)
    pltpu.sync_copy(g_hbm, pg)
    invTs = [pt[pl.ds(b * L, L)] for b in range(NG)]
    gs = [pg[pl.ds(b * L, L)] for b in range(NG)]

    base = wid * SLICE
    xbufs = (xb0, xb1)
    cbufs = (cb0, cb1)
    sems = (sem0, sem1)
    nfull = SLICE // CW
    chunks = [(base, k * CW, CW) for k in range(nfull)]
    if SLICE % CW:
        chunks.append((base, nfull * CW, SLICE % CW))

    def start(k, buf):
        b0, off, w = chunks[k]
        voff = pl.multiple_of(b0 + off, 8)
        hx = pltpu.async_copy(logits_hbm.at[pl.ds(voff, w), :],
                              xbufs[buf].at[pl.ds(0, w), :], sems[buf])
        hc = pltpu.async_copy(c_hbm.at[pl.ds(voff, w), :],
                              cbufs[buf].at[pl.ds(0, w), :], sems[buf])
        return hx, hc

    handles = [None, None]
    handles[0] = start(0, 0)

    ninf = jnp.full((L,), -jnp.inf, jnp.float32)
    zero = jnp.zeros((L,), jnp.int32)
    bvs = [ninf] * NG
    bis = [zero] * NG

    for k in range(len(chunks)):
        cur = k % 2
        hx, hc = handles[cur]
        hx.wait()
        hc.wait()
        if k + 1 < len(chunks):
            handles[1 - cur] = start(k + 1, 1 - cur)
        xref, cref = xbufs[cur], cbufs[cur]
        b0, off, w = chunks[k]
        v0 = b0 + off

        def body(i, carry, xref=xref, cref=cref):
            st = list(carry[:2 * NG])
            iv = carry[2 * NG]
            for b in range(NG):
                x = xref[i, pl.ds(b * L, L)]
                c = cref[i, pl.ds(b * L, L)]
                s = x * invTs[b] + gs[b] * c
                m = s > st[b]
                st[b] = jnp.where(m, s, st[b])
                st[NG + b] = jnp.where(m, iv, st[NG + b])
            return tuple(st) + (iv + 1,)

        iv0 = jnp.broadcast_to(jnp.int32(0), (L,)) + v0
        out = lax.fori_loop(0, w, body, tuple(bvs) + tuple(bis) + (iv0,))
        bvs = list(out[:NG])
        bis = list(out[NG:2 * NG])

    for b in range(NG):
        sbf[pl.ds(b * L, L)] = bvs[b]
        sbi[pl.ds(b * L, L)] = bis[b]
    pltpu.sync_copy(sbf, shf.at[pl.ds(sid * R, R)])
    pltpu.sync_copy(sbi, shi.at[pl.ds(sid * R, R)])
    plsc.subcore_barrier()

    @pl.when(sid == 0)
    def _():
        pltpu.sync_copy(shf, mfb)
        pltpu.sync_copy(shi, mib)
        for b in range(NG):
            av = mfb[pl.ds(b * L, L)]
            ai = mib[pl.ds(b * L, L)]
            for s2 in range(1, 16):
                ov = mfb[pl.ds(s2 * R + b * L, L)]
                oi = mib[pl.ds(s2 * R + b * L, L)]
                m = (ov > av) | ((ov == av) & (oi < ai))
                av = jnp.where(m, ov, av)
                ai = jnp.where(m, oi, ai)
            vbuf[b, pl.ds(0, L)] = av
            ibuf[b, pl.ds(0, L)] = ai
        ro = pl.multiple_of(cid * NG, 8)
        pltpu.sync_copy(vbuf, outv_hbm.at[pl.ds(ro, NG), :])
        pltpu.sync_copy(ibuf, outi_hbm.at[pl.ds(ro, NG), :])


_sampler = functools.partial(
    pl.kernel,
    out_type=(jax.ShapeDtypeStruct((2 * NG, 128), jnp.float32),
              jax.ShapeDtypeStruct((2 * NG, 128), jnp.int32)),
    mesh=plsc.VectorSubcoreMesh(core_axis_name="c", subcore_axis_name="s"),
    compiler_params=pltpu.CompilerParams(use_tc_tiling_on_sc=True),
    scratch_types=[
        pltpu.VMEM((CW, 128), jnp.float32),
        pltpu.VMEM((CW, 128), jnp.float32),
        pltpu.VMEM((CW, 128), jnp.float32),
        pltpu.VMEM((CW, 128), jnp.float32),
        pltpu.VMEM((R,), jnp.float32),
        pltpu.VMEM((R,), jnp.float32),
        pltpu.VMEM((R,), jnp.float32),
        pltpu.VMEM((R,), jnp.int32),
        pltpu.VMEM((16 * R,), jnp.float32),
        pltpu.VMEM((16 * R,), jnp.int32),
        pltpu.VMEM((NG, 128), jnp.float32),
        pltpu.VMEM((NG, 128), jnp.int32),
        pltpu.VMEM_SHARED((16 * R,), jnp.float32),
        pltpu.VMEM_SHARED((16 * R,), jnp.int32),
        pltpu.SemaphoreType.DMA,
        pltpu.SemaphoreType.DMA,
    ],
)(_sampler_body)


def _tc_body(invt_ref, g_ref, x_ref, c_ref, outv_ref, outi_ref, rv, ri):
    k = pl.program_id(0)

    @pl.when(k == 0)
    def _():
        rv[...] = jnp.full((NACC * 8, 128), -jnp.inf, jnp.float32)
        ri[...] = jnp.zeros((NACC * 8, 128), jnp.int32)

    s = x_ref[...] * invt_ref[...] + g_ref[...] * c_ref[...]
    base = SCV + k * TCB
    rows = jax.lax.broadcasted_iota(jnp.int32, (TCB, 128), 0) + base
    rvv = [rv[pl.ds(a * 8, 8), :] for a in range(NACC)]
    riv = [ri[pl.ds(a * 8, 8), :] for a in range(NACC)]
    for j in range(TCB // 8):
        a = j % NACC
        sub = s[j * 8:(j + 1) * 8]
        idx = rows[j * 8:(j + 1) * 8]
        m = (sub > rvv[a]) & (idx < V)
        rvv[a] = jnp.where(m, sub, rvv[a])
        riv[a] = jnp.where(m, idx, riv[a])
    for a in range(NACC):
        rv[pl.ds(a * 8, 8), :] = rvv[a]
        ri[pl.ds(a * 8, 8), :] = riv[a]

    @pl.when(k == TCN - 1)
    def _():
        fv, fi = rvv[0], riv[0]
        for a in range(1, NACC):
            m = (rvv[a] > fv) | ((rvv[a] == fv) & (riv[a] < fi))
            fv = jnp.where(m, rvv[a], fv)
            fi = jnp.where(m, riv[a], fi)
        vmax = jnp.max(fv, axis=0, keepdims=True)
        cand = jnp.where(fv == vmax, fi, V)
        outv_ref[...] = vmax
        outi_ref[...] = jnp.min(cand, axis=0, keepdims=True)


_tc_argmax = pl.pallas_call(
    _tc_body,
    grid=(TCN,),
    in_specs=[
        pl.BlockSpec((1, 128), lambda k: (0, 0)),
        pl.BlockSpec((1, 128), lambda k: (0, 0)),
        pl.BlockSpec((TCB, 128), lambda k: (SCV // TCB + k, 0)),
        pl.BlockSpec((TCB, 128), lambda k: (k, 0)),
    ],
    out_specs=[
        pl.BlockSpec((1, 128), lambda k: (0, 0)),
        pl.BlockSpec((1, 128), lambda k: (0, 0)),
    ],
    out_shape=[
        jax.ShapeDtypeStruct((1, 128), jnp.float32),
        jax.ShapeDtypeStruct((1, 128), jnp.int32),
    ],
    scratch_shapes=[
        pltpu.VMEM((NACC * 8, 128), jnp.float32),
        pltpu.VMEM((NACC * 8, 128), jnp.int32),
    ],
)


def _lex_merge(v0, i0, v1, i1):
    take = (v1 > v0) | ((v1 == v0) & (i1 < i0))
    return jnp.where(take, v1, v0), jnp.where(take, i1, i0)


def kernel(logits, temperatures):
    c_sc, c_tc = _const_table()
    sampled = temperatures > 1e-10
    inv_t = jnp.where(sampled, 1.0 / jnp.where(sampled, temperatures, 1.0), 1.0)
    g = sampled.astype(jnp.float32)
    lt = logits.astype(jnp.float32).T
    outv, outi = _sampler(lt, c_sc, inv_t, g)
    tcv, tci = _tc_argmax(inv_t[None, :], g[None, :], lt, c_tc)
    mv, mi = _lex_merge(outv[0:NG, 0:L].reshape(R), outi[0:NG, 0:L].reshape(R),
                        outv[NG:, 0:L].reshape(R), outi[NG:, 0:L].reshape(R))
    _, ti = _lex_merge(mv, mi, tcv.reshape(R), tci.reshape(R))
    return ti
